# baseline (speedup 1.0000x reference)
import functools
import math

import jax
import jax.numpy as jnp
from jax import lax
from jax.experimental import pallas as pl
from jax.experimental.pallas import tpu as pltpu
from jax.experimental.pallas import tpu_sc as plsc

B = 4096
D = 512
NUM_CLASSES = 100000
M = 0.1
S = 1.0
EPSILON = 0.1

LOG2E = 1.4426950408889634
CS = S * LOG2E

ROW_TILE = 2048
CNT_TILE = 512


def _counts_kernel(lbl_row_ref, invc_ref, k_ref, lblc_ref):
    j = pl.program_id(0)

    @pl.when(j == 0)
    def _():
        lblc_ref[...] = lbl_row_ref[...].reshape(B, 1)

    eq = jnp.where(lblc_ref[...] == lbl_row_ref[0:1, pl.ds(j * CNT_TILE,
                                                           CNT_TILE)], 1.0,
                   0.0)
    invc = 1.0 / jnp.sum(eq, axis=0, keepdims=True)
    invc_ref[...] = invc
    kp = jnp.sum(invc, keepdims=True).reshape(1, 1)

    @pl.when(j == 0)
    def _():
        k_ref[...] = kp

    @pl.when(j != 0)
    def _():
        k_ref[...] += kp


def _loss_kernel(x_ref, cn_ref, invc_ref, k_ref, out_ref, cnbf_ref, v_ref):
    i = pl.program_id(0)

    @pl.when(i == 0)
    def _():
        c = cn_ref[...]
        rn = 1.0 / jnp.sqrt(jnp.sum(c * c, axis=1, keepdims=True))
        cnbf = (c * rn).astype(jnp.bfloat16)
        cnbf_ref[...] = cnbf
        v_ref[...] = jax.lax.dot_general(
            invc_ref[...].astype(jnp.bfloat16), cnbf,
            (((1,), (0,)), ((), ())),
            preferred_element_type=jnp.float32)

    k = k_ref[...]
    inv_c = invc_ref[...]

    xt = x_ref[...]
    rs = 1.0 / jnp.maximum(jnp.sqrt(jnp.sum(xt * xt, axis=1, keepdims=True)),
                           1e-12)
    xs = xt * (rs * CS)

    cnt = cnbf_ref[pl.ds(i * ROW_TILE, ROW_TILE), :].astype(jnp.float32)
    t = jnp.sum(xs * cnt, axis=1, keepdims=True) * (1.0 / CS)
    t = jnp.clip(t, -1.0, 1.0)
    tm = t * math.cos(M) - jnp.sqrt(jnp.maximum(1.0 - t * t, 0.0)) * math.sin(M)

    a2 = jax.lax.dot_general(xs.astype(jnp.bfloat16), cnbf_ref[...],
                             (((1,), (1,)), ((), ())),
                             preferred_element_type=jnp.float32)
    sumexp = jnp.sum(jnp.exp2(a2) * inv_c, axis=1, keepdims=True)
    sumw = jnp.sum(xs * v_ref[...], axis=1, keepdims=True) * (S / CS)

    z = sumexp - jnp.exp2(CS * t) + jnp.exp2(CS * tm)
    sw = sumw - S * t + S * tm
    logz = jnp.log(z)

    row_loss = (-(1.0 - EPSILON) * (S * tm - logz)
                - (EPSILON / NUM_CLASSES) * (sw - k * logz))
    partial = jnp.sum(row_loss, keepdims=True).reshape(1, 1) * (1.0 / B)

    @pl.when(i == 0)
    def _():
        out_ref[...] = partial

    @pl.when(i != 0)
    def _():
        out_ref[...] += partial


def _sc_gather(labels, W):
    info = plsc.get_sparse_core_info()
    nw = info.num_cores * info.num_subcores
    b_per_w = B // nw
    mesh = plsc.VectorSubcoreMesh(core_axis_name="c", subcore_axis_name="s")

    nchunk = 4
    rows_per_chunk = b_per_w // nchunk

    @functools.partial(
        pl.kernel, mesh=mesh,
        out_type=jax.ShapeDtypeStruct((B, D), jnp.float32),
        scratch_types=(
            [pltpu.VMEM((b_per_w,), jnp.int32)]
            + [pltpu.VMEM((rows_per_chunk, D), jnp.float32)] * nchunk
            + [pltpu.SemaphoreType.DMA] * (2 * nchunk)
        ),
    )
    def gather(table_hbm, idx_hbm, out_hbm, idx_v, *bufs_and_sems):
        bufs = bufs_and_sems[:nchunk]
        gsems = bufs_and_sems[nchunk:2 * nchunk]
        wsems = bufs_and_sems[2 * nchunk:]
        wid = lax.axis_index("s") * info.num_cores + lax.axis_index("c")
        base = wid * b_per_w
        pltpu.sync_copy(idx_hbm.at[pl.ds(base, b_per_w)], idx_v)
        gets = []
        for b in range(nchunk):
            gets.append(pltpu.async_copy(
                table_hbm.at[idx_v.at[pl.ds(b * rows_per_chunk,
                                            rows_per_chunk)]],
                bufs[b], gsems[b]))
        puts = []
        for b in range(nchunk):
            gets[b].wait()
            puts.append(pltpu.async_copy(
                bufs[b],
                out_hbm.at[pl.ds(base + b * rows_per_chunk, rows_per_chunk)],
                wsems[b]))
        for b in range(nchunk):
            puts[b].wait()

    return gather(W, labels)


def kernel(x, labels, W):
    cn = _sc_gather(labels, W)

    lbl_row = labels.reshape(1, B)
    invc, kk = pl.pallas_call(
        _counts_kernel,
        grid=(B // CNT_TILE,),
        in_specs=[
            pl.BlockSpec((1, B), lambda j: (0, 0)),
        ],
        out_specs=[
            pl.BlockSpec((1, CNT_TILE), lambda j: (0, j)),
            pl.BlockSpec((1, 1), lambda j: (0, 0)),
        ],
        out_shape=[jax.ShapeDtypeStruct((1, B), jnp.float32),
                   jax.ShapeDtypeStruct((1, 1), jnp.float32)],
        scratch_shapes=[pltpu.VMEM((B, 1), jnp.int32)],
    )(lbl_row)

    out = pl.pallas_call(
        _loss_kernel,
        grid=(B // ROW_TILE,),
        in_specs=[
            pl.BlockSpec((ROW_TILE, D), lambda i: (i, 0)),
            pl.BlockSpec((B, D), lambda i: (0, 0)),
            pl.BlockSpec((1, B), lambda i: (0, 0)),
            pl.BlockSpec((1, 1), lambda i: (0, 0)),
        ],
        out_specs=pl.BlockSpec((1, 1), lambda i: (0, 0)),
        out_shape=jax.ShapeDtypeStruct((1, 1), jnp.float32),
        scratch_shapes=[pltpu.VMEM((B, D), jnp.bfloat16),
                        pltpu.VMEM((1, D), jnp.float32)],
    )(x, cn, invc, kk)
    return out[0, 0]


# speedup vs baseline: 1.0133x; 1.0133x over previous
import functools
import math

import jax
import jax.numpy as jnp
from jax import lax
from jax.experimental import pallas as pl
from jax.experimental.pallas import tpu as pltpu
from jax.experimental.pallas import tpu_sc as plsc

B = 4096
D = 512
NUM_CLASSES = 100000
M = 0.1
S = 1.0
EPSILON = 0.1

LOG2E = 1.4426950408889634
CS = S * LOG2E

ROW_TILE = 1024
CNT_TILE = 512


def _counts_kernel(lbl_row_ref, invc_ref, k_ref, lblc_ref):
    j = pl.program_id(0)

    @pl.when(j == 0)
    def _():
        lblc_ref[...] = lbl_row_ref[...].reshape(B, 1)

    eq = jnp.where(lblc_ref[...] == lbl_row_ref[0:1, pl.ds(j * CNT_TILE,
                                                           CNT_TILE)], 1.0,
                   0.0)
    invc = 1.0 / jnp.sum(eq, axis=0, keepdims=True)
    invc_ref[...] = invc
    kp = jnp.sum(invc, keepdims=True).reshape(1, 1)

    @pl.when(j == 0)
    def _():
        k_ref[...] = kp

    @pl.when(j != 0)
    def _():
        k_ref[...] += kp


def _loss_kernel(x_ref, cn_ref, invc_ref, k_ref, out_ref, cnbf_ref, v_ref):
    i = pl.program_id(0)

    @pl.when(i == 0)
    def _():
        c = cn_ref[...]
        rn = 1.0 / jnp.sqrt(jnp.sum(c * c, axis=1, keepdims=True))
        cnbf = (c * rn).astype(jnp.bfloat16)
        cnbf_ref[...] = cnbf
        v_ref[...] = jax.lax.dot_general(
            invc_ref[...].astype(jnp.bfloat16), cnbf,
            (((1,), (0,)), ((), ())),
            preferred_element_type=jnp.float32)

    k = k_ref[...]
    inv_c = invc_ref[...]

    xt = x_ref[...]
    rs = 1.0 / jnp.maximum(jnp.sqrt(jnp.sum(xt * xt, axis=1, keepdims=True)),
                           1e-12)
    xs = xt * (rs * CS)

    cnt = cnbf_ref[pl.ds(i * ROW_TILE, ROW_TILE), :].astype(jnp.float32)
    t = jnp.sum(xs * cnt, axis=1, keepdims=True) * (1.0 / CS)
    t = jnp.clip(t, -1.0, 1.0)
    tm = t * math.cos(M) - jnp.sqrt(jnp.maximum(1.0 - t * t, 0.0)) * math.sin(M)

    a2 = jax.lax.dot_general(xs.astype(jnp.bfloat16), cnbf_ref[...],
                             (((1,), (1,)), ((), ())),
                             preferred_element_type=jnp.float32)
    sumexp = jnp.sum(jnp.exp2(a2) * inv_c, axis=1, keepdims=True)
    sumw = jnp.sum(xs * v_ref[...], axis=1, keepdims=True) * (S / CS)

    z = sumexp - jnp.exp2(CS * t) + jnp.exp2(CS * tm)
    sw = sumw - S * t + S * tm
    logz = jnp.log(z)

    row_loss = (-(1.0 - EPSILON) * (S * tm - logz)
                - (EPSILON / NUM_CLASSES) * (sw - k * logz))
    partial = jnp.sum(row_loss, keepdims=True).reshape(1, 1) * (1.0 / B)

    @pl.when(i == 0)
    def _():
        out_ref[...] = partial

    @pl.when(i != 0)
    def _():
        out_ref[...] += partial


def _sc_gather(labels, W):
    info = plsc.get_sparse_core_info()
    nw = info.num_cores * info.num_subcores
    b_per_w = B // nw
    mesh = plsc.VectorSubcoreMesh(core_axis_name="c", subcore_axis_name="s")

    nchunk = 4
    rows_per_chunk = b_per_w // nchunk

    @functools.partial(
        pl.kernel, mesh=mesh,
        out_type=jax.ShapeDtypeStruct((B, D), jnp.float32),
        scratch_types=(
            [pltpu.VMEM((b_per_w,), jnp.int32)]
            + [pltpu.VMEM((rows_per_chunk, D), jnp.float32)] * nchunk
            + [pltpu.SemaphoreType.DMA] * (2 * nchunk)
        ),
    )
    def gather(table_hbm, idx_hbm, out_hbm, idx_v, *bufs_and_sems):
        bufs = bufs_and_sems[:nchunk]
        gsems = bufs_and_sems[nchunk:2 * nchunk]
        wsems = bufs_and_sems[2 * nchunk:]
        wid = lax.axis_index("s") * info.num_cores + lax.axis_index("c")
        base = wid * b_per_w
        pltpu.sync_copy(idx_hbm.at[pl.ds(base, b_per_w)], idx_v)
        gets = []
        for b in range(nchunk):
            gets.append(pltpu.async_copy(
                table_hbm.at[idx_v.at[pl.ds(b * rows_per_chunk,
                                            rows_per_chunk)]],
                bufs[b], gsems[b]))
        puts = []
        for b in range(nchunk):
            gets[b].wait()
            puts.append(pltpu.async_copy(
                bufs[b],
                out_hbm.at[pl.ds(base + b * rows_per_chunk, rows_per_chunk)],
                wsems[b]))
        for b in range(nchunk):
            puts[b].wait()

    return gather(W, labels)


def kernel(x, labels, W):
    cn = _sc_gather(labels, W)

    lbl_row = labels.reshape(1, B)
    invc, kk = pl.pallas_call(
        _counts_kernel,
        grid=(B // CNT_TILE,),
        in_specs=[
            pl.BlockSpec((1, B), lambda j: (0, 0)),
        ],
        out_specs=[
            pl.BlockSpec((1, CNT_TILE), lambda j: (0, j)),
            pl.BlockSpec((1, 1), lambda j: (0, 0)),
        ],
        out_shape=[jax.ShapeDtypeStruct((1, B), jnp.float32),
                   jax.ShapeDtypeStruct((1, 1), jnp.float32)],
        scratch_shapes=[pltpu.VMEM((B, 1), jnp.int32)],
    )(lbl_row)

    out = pl.pallas_call(
        _loss_kernel,
        grid=(B // ROW_TILE,),
        in_specs=[
            pl.BlockSpec((ROW_TILE, D), lambda i: (i, 0)),
            pl.BlockSpec((B, D), lambda i: (0, 0)),
            pl.BlockSpec((1, B), lambda i: (0, 0)),
            pl.BlockSpec((1, 1), lambda i: (0, 0)),
        ],
        out_specs=pl.BlockSpec((1, 1), lambda i: (0, 0)),
        out_shape=jax.ShapeDtypeStruct((1, 1), jnp.float32),
        scratch_shapes=[pltpu.VMEM((B, D), jnp.bfloat16),
                        pltpu.VMEM((1, D), jnp.float32)],
    )(x, cn, invc, kk)
    return out[0, 0]


# speedup vs baseline: 1.0148x; 1.0015x over previous
import functools
import math

import jax
import jax.numpy as jnp
from jax import lax
from jax.experimental import pallas as pl
from jax.experimental.pallas import tpu as pltpu
from jax.experimental.pallas import tpu_sc as plsc

B = 4096
D = 512
NUM_CLASSES = 100000
M = 0.1
S = 1.0
EPSILON = 0.1

LOG2E = 1.4426950408889634
CS = S * LOG2E

ROW_TILE = 1024
CNT_TILE = 512


def _counts_kernel(lbl_row_ref, invc_ref, k_ref, lblc_ref):
    j = pl.program_id(0)

    @pl.when(j == 0)
    def _():
        lblc_ref[...] = lbl_row_ref[...].reshape(B, 1)

    eq = jnp.where(lblc_ref[...] == lbl_row_ref[0:1, pl.ds(j * CNT_TILE,
                                                           CNT_TILE)], 1.0,
                   0.0)
    invc = 1.0 / jnp.sum(eq, axis=0, keepdims=True)
    invc_ref[...] = invc
    kp = jnp.sum(invc, keepdims=True).reshape(1, 1)

    @pl.when(j == 0)
    def _():
        k_ref[...] = kp

    @pl.when(j != 0)
    def _():
        k_ref[...] += kp


def _loss_kernel(x_ref, cn_ref, invc_ref, k_ref, out_ref, cnbf_ref, v_ref):
    i = pl.program_id(0)

    @pl.when(i == 0)
    def _():
        c = cn_ref[...]
        rn = 1.0 / jnp.sqrt(jnp.sum(c * c, axis=1, keepdims=True))
        cnbf = (c * rn).astype(jnp.bfloat16)
        cnbf_ref[...] = cnbf
        v_ref[...] = jax.lax.dot_general(
            invc_ref[...].astype(jnp.bfloat16), cnbf,
            (((1,), (0,)), ((), ())),
            preferred_element_type=jnp.float32)

    k = k_ref[...]
    inv_c = invc_ref[...]

    xt = x_ref[...]
    rs = 1.0 / jnp.maximum(jnp.sqrt(jnp.sum(xt * xt, axis=1, keepdims=True)),
                           1e-12)
    xs = xt * (rs * CS)

    cnt = cnbf_ref[pl.ds(i * ROW_TILE, ROW_TILE), :].astype(jnp.float32)
    t = jnp.sum(xs * cnt, axis=1, keepdims=True) * (1.0 / CS)
    t = jnp.clip(t, -1.0, 1.0)
    tm = t * math.cos(M) - jnp.sqrt(jnp.maximum(1.0 - t * t, 0.0)) * math.sin(M)

    a2 = jax.lax.dot_general(xs.astype(jnp.bfloat16), cnbf_ref[...],
                             (((1,), (1,)), ((), ())),
                             preferred_element_type=jnp.float32)
    sumexp = jnp.sum(jnp.exp2(a2) * inv_c, axis=1, keepdims=True)
    sumw = jnp.sum(xs * v_ref[...], axis=1, keepdims=True) * (S / CS)

    z = sumexp - jnp.exp2(CS * t) + jnp.exp2(CS * tm)
    sw = sumw - S * t + S * tm
    logz = jnp.log(z)

    row_loss = (-(1.0 - EPSILON) * (S * tm - logz)
                - (EPSILON / NUM_CLASSES) * (sw - k * logz))
    partial = jnp.sum(row_loss, keepdims=True).reshape(1, 1) * (1.0 / B)

    @pl.when(i == 0)
    def _():
        out_ref[...] = partial

    @pl.when(i != 0)
    def _():
        out_ref[...] += partial


def _sc_gather(labels, W):
    info = plsc.get_sparse_core_info()
    nw = info.num_cores * info.num_subcores
    b_per_w = B // nw
    mesh = plsc.VectorSubcoreMesh(core_axis_name="c", subcore_axis_name="s")

    nchunk = 8
    rows_per_chunk = b_per_w // nchunk

    @functools.partial(
        pl.kernel, mesh=mesh,
        out_type=jax.ShapeDtypeStruct((B, D), jnp.float32),
        scratch_types=(
            [pltpu.VMEM((b_per_w,), jnp.int32)]
            + [pltpu.VMEM((rows_per_chunk, D), jnp.float32)] * nchunk
            + [pltpu.SemaphoreType.DMA] * (2 * nchunk)
        ),
    )
    def gather(table_hbm, idx_hbm, out_hbm, idx_v, *bufs_and_sems):
        bufs = bufs_and_sems[:nchunk]
        gsems = bufs_and_sems[nchunk:2 * nchunk]
        wsems = bufs_and_sems[2 * nchunk:]
        wid = lax.axis_index("s") * info.num_cores + lax.axis_index("c")
        base = wid * b_per_w
        pltpu.sync_copy(idx_hbm.at[pl.ds(base, b_per_w)], idx_v)
        gets = []
        for b in range(nchunk):
            gets.append(pltpu.async_copy(
                table_hbm.at[idx_v.at[pl.ds(b * rows_per_chunk,
                                            rows_per_chunk)]],
                bufs[b], gsems[b]))
        puts = []
        for b in range(nchunk):
            gets[b].wait()
            puts.append(pltpu.async_copy(
                bufs[b],
                out_hbm.at[pl.ds(base + b * rows_per_chunk, rows_per_chunk)],
                wsems[b]))
        for b in range(nchunk):
            puts[b].wait()

    return gather(W, labels)


def kernel(x, labels, W):
    cn = _sc_gather(labels, W)

    lbl_row = labels.reshape(1, B)
    invc, kk = pl.pallas_call(
        _counts_kernel,
        grid=(B // CNT_TILE,),
        in_specs=[
            pl.BlockSpec((1, B), lambda j: (0, 0)),
        ],
        out_specs=[
            pl.BlockSpec((1, CNT_TILE), lambda j: (0, j)),
            pl.BlockSpec((1, 1), lambda j: (0, 0)),
        ],
        out_shape=[jax.ShapeDtypeStruct((1, B), jnp.float32),
                   jax.ShapeDtypeStruct((1, 1), jnp.float32)],
        scratch_shapes=[pltpu.VMEM((B, 1), jnp.int32)],
    )(lbl_row)

    out = pl.pallas_call(
        _loss_kernel,
        grid=(B // ROW_TILE,),
        in_specs=[
            pl.BlockSpec((ROW_TILE, D), lambda i: (i, 0)),
            pl.BlockSpec((B, D), lambda i: (0, 0)),
            pl.BlockSpec((1, B), lambda i: (0, 0)),
            pl.BlockSpec((1, 1), lambda i: (0, 0)),
        ],
        out_specs=pl.BlockSpec((1, 1), lambda i: (0, 0)),
        out_shape=jax.ShapeDtypeStruct((1, 1), jnp.float32),
        scratch_shapes=[pltpu.VMEM((B, D), jnp.bfloat16),
                        pltpu.VMEM((1, D), jnp.float32)],
    )(x, cn, invc, kk)
    return out[0, 0]
